# baseline (speedup 1.0000x reference)
import functools

import jax
import jax.numpy as jnp
from jax import lax
from jax.experimental import pallas as pl
from jax.experimental.pallas import tpu as pltpu
from jax.experimental.pallas import tpu_sc as plsc

TOPK = 32


def _matvec_body(b_ref, w_ref, d_ref):
    bb = b_ref[0].astype(jnp.bfloat16).astype(jnp.float32)
    ww = w_ref[0].astype(jnp.bfloat16).astype(jnp.float32)
    d_ref[0, 0] = jnp.sum(bb * ww, axis=1)


@functools.lru_cache(maxsize=None)
def _make_matvec(B, S, H, BS):
    nblk = S // BS
    return pl.pallas_call(
        _matvec_body,
        grid=(B, nblk),
        in_specs=[
            pl.BlockSpec((1, BS, H), lambda b, j: (b, j, 0)),
            pl.BlockSpec((1, 1, H), lambda b, j: (b, 0, 0)),
        ],
        out_specs=pl.BlockSpec((1, 1, BS), lambda b, j: (b * nblk + j, 0, 0)),
        out_shape=jax.ShapeDtypeStruct((B * nblk, 1, BS), jnp.float32),
    )


@functools.lru_cache(maxsize=None)
def _make_sc_gather(B, S):
    info = plsc.get_sparse_core_info()
    NC, NS = info.num_cores, info.num_subcores
    NW = NC * NS
    per_b = NW // B
    b_per_c = B // NC
    CH = S // per_b
    R = CH // 128
    mesh = plsc.VectorSubcoreMesh(core_axis_name="c", subcore_axis_name="s")

    @functools.partial(
        pl.kernel,
        out_type=(
            jax.ShapeDtypeStruct((B, S), jnp.float32),
            jax.ShapeDtypeStruct((B, S), jnp.float32),
        ),
        mesh=mesh,
        compiler_params=pltpu.CompilerParams(needs_layout_passes=False),
        scratch_types=[
            pltpu.VMEM((S,), jnp.float32),
            pltpu.VMEM((CH,), jnp.int32),
            pltpu.VMEM((CH,), jnp.float32),
            pltpu.VMEM((R, 128), jnp.int32),
            pltpu.VMEM((128,), jnp.float32),
            pltpu.VMEM((CH,), jnp.float32),
            pltpu.VMEM_SHARED((b_per_c * S,), jnp.float32),
        ],
    )
    def sc_fn(d_hbm, pos_hbm, det_hbm, cnt_hbm,
              d_v, pos_v, det_v, idx_v, ones_v, tmp_v, cnt_sh):
        c = lax.axis_index("c")
        s = lax.axis_index("s")
        lb = s // per_b
        b = c * b_per_c + lb
        base_s = (s - lb * per_b) * CH

        pltpu.sync_copy(d_hbm.at[b], d_v)
        pltpu.sync_copy(pos_hbm.at[b, pl.ds(base_s, CH)], pos_v)

        one16 = jnp.ones((16,), jnp.float32)
        zero16 = jnp.zeros((16,), jnp.float32)
        boff = lb * S
        for i in range(8):
            ones_v[pl.ds(i * 16, 16)] = one16
        for i in range(CH // 16):
            sl = pl.ds(i * 16, 16)
            idx = pos_v[sl]
            val = plsc.load_gather(d_v, [idx])
            det_v[sl] = jnp.maximum(val, 0.0)
            idx_v[i // 8, pl.ds((i % 8) * 16, 16)] = idx + boff
            tmp_v[sl] = zero16

        pltpu.sync_copy(tmp_v, cnt_sh.at[pl.ds(s * CH, CH)])
        plsc.subcore_barrier()
        for r in range(R):
            pltpu.sync_copy(ones_v, cnt_sh.at[idx_v.at[r]], add=True)
        plsc.subcore_barrier()
        pltpu.sync_copy(cnt_sh.at[pl.ds(s * CH, CH)], tmp_v)
        pltpu.sync_copy(tmp_v, cnt_hbm.at[b, pl.ds(base_s, CH)])
        pltpu.sync_copy(det_v, det_hbm.at[b, pl.ds(base_s, CH)])

    return sc_fn


def _final_body(det_ref, b_ref, c_ref, w_ref, o_ref, nt_ref, steer_ref):
    j = pl.program_id(1)
    S = det_ref.shape[2]

    @pl.when(j == 0)
    def _():
        v = det_ref[0]
        iota = lax.broadcasted_iota(jnp.int32, (1, S), 1)
        nt = v
        tot = jnp.zeros((1, 1), jnp.float32)
        for _ in range(TOPK):
            m = jnp.max(v, axis=1, keepdims=True)
            first = jnp.min(jnp.where(v == m, iota, S), axis=1, keepdims=True)
            hit = iota == first
            nt = jnp.where(hit, 0.0, nt)
            v = jnp.where(hit, -1.0, v)
            tot = tot + m
        nt_ref[0] = nt
        steer_ref[...] = (tot / TOPK) * w_ref[0]

    cnt = c_ref[0, 0]
    mask = (cnt > 0.0).astype(jnp.float32)
    o_ref[0] = b_ref[0] + mask[:, None] * steer_ref[...]


@functools.lru_cache(maxsize=None)
def _make_final(B, S, H, BS):
    nblk = S // BS
    return pl.pallas_call(
        _final_body,
        grid=(B, nblk),
        in_specs=[
            pl.BlockSpec((1, 1, S), lambda b, j: (b, 0, 0)),
            pl.BlockSpec((1, BS, H), lambda b, j: (b, j, 0)),
            pl.BlockSpec((1, 1, BS), lambda b, j: (b * nblk + j, 0, 0)),
            pl.BlockSpec((1, 1, H), lambda b, j: (b, 0, 0)),
        ],
        out_specs=(
            pl.BlockSpec((1, BS, H), lambda b, j: (b, j, 0)),
            pl.BlockSpec((1, 1, S), lambda b, j: (b, 0, 0)),
        ),
        out_shape=(
            jax.ShapeDtypeStruct((B, S, H), jnp.float32),
            jax.ShapeDtypeStruct((B, 1, S), jnp.float32),
        ),
        scratch_shapes=[pltpu.VMEM((1, H), jnp.float32)],
    )


def kernel(base, intervention_positions, batch_weights):
    B, S, H = base.shape
    BS = 512
    nblk = S // BS

    d3 = _make_matvec(B, S, H, BS)(base, batch_weights)
    d = d3.reshape(B, S)

    det, cnt = _make_sc_gather(B, S)(d, intervention_positions)

    out, nt = _make_final(B, S, H, BS)(
        det.reshape(B, 1, S),
        base,
        cnt.reshape(B * nblk, 1, BS),
        batch_weights,
    )
    return out, det, nt.reshape(B, S)


# speedup vs baseline: 1.1782x; 1.1782x over previous
import functools

import jax
import jax.numpy as jnp
from jax import lax
from jax.experimental import pallas as pl
from jax.experimental.pallas import tpu as pltpu
from jax.experimental.pallas import tpu_sc as plsc

TOPK = 32


def _matvec_body(b_ref, w_ref, d_ref):
    bb = b_ref[0].astype(jnp.bfloat16).astype(jnp.float32)
    ww = w_ref[0].astype(jnp.bfloat16).astype(jnp.float32)
    d_ref[0, 0] = jnp.sum(bb * ww, axis=1)


@functools.lru_cache(maxsize=None)
def _make_matvec(B, S, H, BS):
    nblk = S // BS
    return pl.pallas_call(
        _matvec_body,
        grid=(B, nblk),
        in_specs=[
            pl.BlockSpec((1, BS, H), lambda b, j: (b, j, 0)),
            pl.BlockSpec((1, 1, H), lambda b, j: (b, 0, 0)),
        ],
        out_specs=pl.BlockSpec((1, 1, BS), lambda b, j: (b * nblk + j, 0, 0)),
        out_shape=jax.ShapeDtypeStruct((B * nblk, 1, BS), jnp.float32),
    )


@functools.lru_cache(maxsize=None)
def _make_sc_gather(B, S):
    info = plsc.get_sparse_core_info()
    NC, NS = info.num_cores, info.num_subcores
    NW = NC * NS
    per_b = NW // B
    b_per_c = B // NC
    CH = S // per_b
    R = CH // 128
    mesh = plsc.VectorSubcoreMesh(core_axis_name="c", subcore_axis_name="s")

    @functools.partial(
        pl.kernel,
        out_type=(
            jax.ShapeDtypeStruct((B, S), jnp.float32),
            jax.ShapeDtypeStruct((B, S), jnp.float32),
        ),
        mesh=mesh,
        compiler_params=pltpu.CompilerParams(needs_layout_passes=False),
        scratch_types=[
            pltpu.VMEM((S,), jnp.float32),
            pltpu.VMEM((CH,), jnp.int32),
            pltpu.VMEM((CH,), jnp.float32),
            pltpu.VMEM((R, 128), jnp.int32),
            pltpu.VMEM((128,), jnp.float32),
            pltpu.VMEM((CH,), jnp.float32),
            pltpu.VMEM_SHARED((b_per_c * S,), jnp.float32),
        ],
    )
    def sc_fn(d_hbm, pos_hbm, det_hbm, cnt_hbm,
              d_v, pos_v, det_v, idx_v, ones_v, tmp_v, cnt_sh):
        c = lax.axis_index("c")
        s = lax.axis_index("s")
        lb = s // per_b
        b = c * b_per_c + lb
        base_s = (s - lb * per_b) * CH

        pltpu.sync_copy(d_hbm.at[b], d_v)
        pltpu.sync_copy(pos_hbm.at[b, pl.ds(base_s, CH)], pos_v)

        one16 = jnp.ones((16,), jnp.float32)
        zero16 = jnp.zeros((16,), jnp.float32)
        boff = lb * S
        for i in range(8):
            ones_v[pl.ds(i * 16, 16)] = one16
        for i in range(CH // 16):
            sl = pl.ds(i * 16, 16)
            idx = pos_v[sl]
            val = plsc.load_gather(d_v, [idx])
            det_v[sl] = jnp.maximum(val, 0.0)
            idx_v[i // 8, pl.ds((i % 8) * 16, 16)] = idx + boff
            tmp_v[sl] = zero16

        pltpu.sync_copy(tmp_v, cnt_sh.at[pl.ds(s * CH, CH)])
        plsc.subcore_barrier()
        for r in range(R):
            pltpu.sync_copy(ones_v, cnt_sh.at[idx_v.at[r]], add=True)
        plsc.subcore_barrier()
        pltpu.sync_copy(cnt_sh.at[pl.ds(s * CH, CH)], tmp_v)
        pltpu.sync_copy(tmp_v, cnt_hbm.at[b, pl.ds(base_s, CH)])
        pltpu.sync_copy(det_v, det_hbm.at[b, pl.ds(base_s, CH)])

    return sc_fn


@functools.lru_cache(maxsize=None)
def _make_topk(B, S):
    def _topk_body(det_ref, nt_ref, sc_ref):
        v = det_ref[...]
        iota = lax.broadcasted_iota(jnp.int32, (B, S), 1)
        nt = v
        tot = jnp.zeros((B, 1), jnp.float32)
        for _ in range(TOPK):
            m = jnp.max(v, axis=1, keepdims=True)
            first = jnp.min(jnp.where(v == m, iota, S), axis=1, keepdims=True)
            hit = iota == first
            nt = jnp.where(hit, 0.0, nt)
            v = jnp.where(hit, -1.0, v)
            tot = tot + m
        nt_ref[...] = nt
        sc_ref[...] = jnp.broadcast_to(tot / TOPK, (B, 128))

    return pl.pallas_call(
        _topk_body,
        out_shape=(
            jax.ShapeDtypeStruct((B, S), jnp.float32),
            jax.ShapeDtypeStruct((B, 128), jnp.float32),
        ),
    )


def _final_body(b_ref, c_ref, w_ref, s_ref, o_ref):
    cnt = c_ref[0, 0]
    steer = s_ref[0, 0, 0] * w_ref[0, 0]
    mask = (cnt > 0.0).astype(jnp.float32)
    o_ref[0] = b_ref[0] + mask[:, None] * steer[None, :]


@functools.lru_cache(maxsize=None)
def _make_final(B, S, H, BS):
    nblk = S // BS
    return pl.pallas_call(
        _final_body,
        grid=(B, nblk),
        in_specs=[
            pl.BlockSpec((1, BS, H), lambda b, j: (b, j, 0)),
            pl.BlockSpec((1, 1, BS), lambda b, j: (b * nblk + j, 0, 0)),
            pl.BlockSpec((1, 1, H), lambda b, j: (b, 0, 0)),
            pl.BlockSpec((1, 1, 128), lambda b, j: (b, 0, 0)),
        ],
        out_specs=pl.BlockSpec((1, BS, H), lambda b, j: (b, j, 0)),
        out_shape=jax.ShapeDtypeStruct((B, S, H), jnp.float32),
    )


def kernel(base, intervention_positions, batch_weights):
    B, S, H = base.shape
    BS = 1024
    nblk = S // BS

    d3 = _make_matvec(B, S, H, BS)(base, batch_weights)
    d = d3.reshape(B, S)

    det, cnt = _make_sc_gather(B, S)(d, intervention_positions)

    nt, scale = _make_topk(B, S)(det)

    out = _make_final(B, S, H, BS)(
        base,
        cnt.reshape(B * nblk, 1, BS),
        batch_weights,
        scale.reshape(B, 1, 128),
    )
    return out, det, nt


# speedup vs baseline: 1.1868x; 1.0074x over previous
import functools

import jax
import jax.numpy as jnp
from jax import lax
from jax.experimental import pallas as pl
from jax.experimental.pallas import tpu as pltpu
from jax.experimental.pallas import tpu_sc as plsc

TOPK = 32


def _matvec_body(b_ref, w_ref, d_ref):
    bb = b_ref[0].astype(jnp.bfloat16).astype(jnp.float32)
    ww = w_ref[0].astype(jnp.bfloat16).astype(jnp.float32)
    d_ref[0, 0] = jnp.sum(bb * ww, axis=1)


@functools.lru_cache(maxsize=None)
def _make_matvec(B, S, H, BS):
    nblk = S // BS
    return pl.pallas_call(
        _matvec_body,
        grid=(B, nblk),
        in_specs=[
            pl.BlockSpec((1, BS, H), lambda b, j: (b, j, 0)),
            pl.BlockSpec((1, 1, H), lambda b, j: (b, 0, 0)),
        ],
        out_specs=pl.BlockSpec((1, 1, BS), lambda b, j: (b * nblk + j, 0, 0)),
        out_shape=jax.ShapeDtypeStruct((B * nblk, 1, BS), jnp.float32),
    )


@functools.lru_cache(maxsize=None)
def _make_sc_gather(B, S):
    info = plsc.get_sparse_core_info()
    NC, NS = info.num_cores, info.num_subcores
    NW = NC * NS
    per_b = NW // B
    b_per_c = B // NC
    CH = S // per_b
    R = CH // 128
    mesh = plsc.VectorSubcoreMesh(core_axis_name="c", subcore_axis_name="s")

    @functools.partial(
        pl.kernel,
        out_type=(
            jax.ShapeDtypeStruct((B, S), jnp.float32),
            jax.ShapeDtypeStruct((B, S), jnp.float32),
        ),
        mesh=mesh,
        compiler_params=pltpu.CompilerParams(needs_layout_passes=False),
        scratch_types=[
            pltpu.VMEM((S,), jnp.float32),
            pltpu.VMEM((CH,), jnp.int32),
            pltpu.VMEM((CH,), jnp.float32),
            pltpu.VMEM((R, 128), jnp.int32),
            pltpu.VMEM((128,), jnp.float32),
            pltpu.VMEM((CH,), jnp.float32),
            pltpu.VMEM_SHARED((b_per_c * S,), jnp.float32),
        ],
    )
    def sc_fn(d_hbm, pos_hbm, det_hbm, cnt_hbm,
              d_v, pos_v, det_v, idx_v, ones_v, tmp_v, cnt_sh):
        c = lax.axis_index("c")
        s = lax.axis_index("s")
        lb = s // per_b
        b = c * b_per_c + lb
        base_s = (s - lb * per_b) * CH

        pltpu.sync_copy(d_hbm.at[b], d_v)
        pltpu.sync_copy(pos_hbm.at[b, pl.ds(base_s, CH)], pos_v)

        one16 = jnp.ones((16,), jnp.float32)
        zero16 = jnp.zeros((16,), jnp.float32)
        boff = lb * S
        for i in range(8):
            ones_v[pl.ds(i * 16, 16)] = one16
        for i in range(CH // 16):
            sl = pl.ds(i * 16, 16)
            idx = pos_v[sl]
            val = plsc.load_gather(d_v, [idx])
            det_v[sl] = jnp.maximum(val, 0.0)
            idx_v[i // 8, pl.ds((i % 8) * 16, 16)] = idx + boff
            tmp_v[sl] = zero16

        pltpu.sync_copy(tmp_v, cnt_sh.at[pl.ds(s * CH, CH)])
        plsc.subcore_barrier()
        for r in range(R):
            pltpu.sync_copy(ones_v, cnt_sh.at[idx_v.at[r]], add=True)
        plsc.subcore_barrier()
        pltpu.sync_copy(cnt_sh.at[pl.ds(s * CH, CH)], tmp_v)
        pltpu.sync_copy(tmp_v, cnt_hbm.at[b, pl.ds(base_s, CH)])
        pltpu.sync_copy(det_v, det_hbm.at[b, pl.ds(base_s, CH)])

    return sc_fn


@functools.lru_cache(maxsize=None)
def _make_topk(B, S):
    def _topk_body(det_ref, nt_ref, sc_ref):
        v = det_ref[...]
        iota = lax.broadcasted_iota(jnp.int32, (B, S), 1)
        nt = v
        tot = jnp.zeros((B, 1), jnp.float32)
        for _ in range(TOPK):
            m = jnp.max(v, axis=1, keepdims=True)
            first = jnp.min(jnp.where(v == m, iota, S), axis=1, keepdims=True)
            hit = iota == first
            nt = jnp.where(hit, 0.0, nt)
            v = jnp.where(hit, -1.0, v)
            tot = tot + m
        nt_ref[...] = nt
        sc_ref[...] = jnp.broadcast_to(tot / TOPK, (B, 128))

    return pl.pallas_call(
        _topk_body,
        out_shape=(
            jax.ShapeDtypeStruct((B, S), jnp.float32),
            jax.ShapeDtypeStruct((B, 128), jnp.float32),
        ),
    )


def _final_body(b_ref, c_ref, w_ref, s_ref, o_ref):
    cnt = c_ref[0, 0]
    steer = s_ref[0, 0, 0] * w_ref[0, 0]
    mask = (cnt > 0.0).astype(jnp.float32)
    o_ref[0] = b_ref[0] + mask[:, None] * steer[None, :]


@functools.lru_cache(maxsize=None)
def _make_final(B, S, H, BS):
    nblk = S // BS
    return pl.pallas_call(
        _final_body,
        grid=(B, nblk),
        in_specs=[
            pl.BlockSpec((1, BS, H), lambda b, j: (b, j, 0)),
            pl.BlockSpec((1, 1, BS), lambda b, j: (b * nblk + j, 0, 0)),
            pl.BlockSpec((1, 1, H), lambda b, j: (b, 0, 0)),
            pl.BlockSpec((1, 1, 128), lambda b, j: (b, 0, 0)),
        ],
        out_specs=pl.BlockSpec((1, BS, H), lambda b, j: (b, j, 0)),
        out_shape=jax.ShapeDtypeStruct((B, S, H), jnp.float32),
    )


def kernel(base, intervention_positions, batch_weights):
    B, S, H = base.shape
    BS1 = 2048
    BS3 = 1024
    nblk3 = S // BS3

    d3 = _make_matvec(B, S, H, BS1)(base, batch_weights)
    d = d3.reshape(B, S)

    det, cnt = _make_sc_gather(B, S)(d, intervention_positions)

    nt, scale = _make_topk(B, S)(det)

    out = _make_final(B, S, H, BS3)(
        base,
        cnt.reshape(B * nblk3, 1, BS3),
        batch_weights,
        scale.reshape(B, 1, 128),
    )
    return out, det, nt


# speedup vs baseline: 1.1982x; 1.0095x over previous
import functools

import jax
import jax.numpy as jnp
from jax import lax
from jax.experimental import pallas as pl
from jax.experimental.pallas import tpu as pltpu
from jax.experimental.pallas import tpu_sc as plsc

TOPK = 32


def _matvec_body(b_ref, w_ref, d_ref):
    bb = b_ref[...].astype(jnp.bfloat16).astype(jnp.float32)
    ww = w_ref[...].astype(jnp.bfloat16).astype(jnp.float32)
    d_ref[...] = jnp.sum(bb * ww, axis=2)


@functools.lru_cache(maxsize=None)
def _make_matvec(B, S, H, BS):
    nblk = S // BS
    return pl.pallas_call(
        _matvec_body,
        grid=(nblk,),
        in_specs=[
            pl.BlockSpec((B, BS, H), lambda j: (0, j, 0)),
            pl.BlockSpec((B, 1, H), lambda j: (0, 0, 0)),
        ],
        out_specs=pl.BlockSpec((B, BS), lambda j: (0, j)),
        out_shape=jax.ShapeDtypeStruct((B, S), jnp.float32),
    )


@functools.lru_cache(maxsize=None)
def _make_sc_gather(B, S):
    info = plsc.get_sparse_core_info()
    NC, NS = info.num_cores, info.num_subcores
    NW = NC * NS
    per_b = NW // B
    b_per_c = B // NC
    CH = S // per_b
    R = CH // 128
    mesh = plsc.VectorSubcoreMesh(core_axis_name="c", subcore_axis_name="s")

    @functools.partial(
        pl.kernel,
        out_type=(
            jax.ShapeDtypeStruct((B, S), jnp.float32),
            jax.ShapeDtypeStruct((B, S), jnp.float32),
        ),
        mesh=mesh,
        compiler_params=pltpu.CompilerParams(needs_layout_passes=False),
        scratch_types=[
            pltpu.VMEM((S,), jnp.float32),
            pltpu.VMEM((CH,), jnp.int32),
            pltpu.VMEM((CH,), jnp.float32),
            pltpu.VMEM((R, 128), jnp.int32),
            pltpu.VMEM((128,), jnp.float32),
            pltpu.VMEM((CH,), jnp.float32),
            pltpu.VMEM_SHARED((b_per_c * S,), jnp.float32),
        ],
    )
    def sc_fn(d_hbm, pos_hbm, det_hbm, cnt_hbm,
              d_v, pos_v, det_v, idx_v, ones_v, tmp_v, cnt_sh):
        c = lax.axis_index("c")
        s = lax.axis_index("s")
        lb = s // per_b
        b = c * b_per_c + lb
        base_s = (s - lb * per_b) * CH

        pltpu.sync_copy(d_hbm.at[b], d_v)
        pltpu.sync_copy(pos_hbm.at[b, pl.ds(base_s, CH)], pos_v)

        one16 = jnp.ones((16,), jnp.float32)
        zero16 = jnp.zeros((16,), jnp.float32)
        boff = lb * S
        for i in range(8):
            ones_v[pl.ds(i * 16, 16)] = one16
        for i in range(CH // 16):
            sl = pl.ds(i * 16, 16)
            idx = pos_v[sl]
            val = plsc.load_gather(d_v, [idx])
            det_v[sl] = jnp.maximum(val, 0.0)
            idx_v[i // 8, pl.ds((i % 8) * 16, 16)] = idx + boff
            tmp_v[sl] = zero16

        pltpu.sync_copy(tmp_v, cnt_sh.at[pl.ds(s * CH, CH)])
        plsc.subcore_barrier()
        for r in range(R):
            pltpu.sync_copy(ones_v, cnt_sh.at[idx_v.at[r]], add=True)
        plsc.subcore_barrier()
        pltpu.sync_copy(cnt_sh.at[pl.ds(s * CH, CH)], tmp_v)
        pltpu.sync_copy(tmp_v, cnt_hbm.at[b, pl.ds(base_s, CH)])
        pltpu.sync_copy(det_v, det_hbm.at[b, pl.ds(base_s, CH)])

    return sc_fn


@functools.lru_cache(maxsize=None)
def _make_topk(B, S):
    def _topk_body(det_ref, nt_ref, sc_ref):
        v = det_ref[...]
        iota = lax.broadcasted_iota(jnp.int32, (B, S), 1)
        tot = jnp.zeros((B, 1), jnp.float32)
        for _ in range(TOPK):
            m = jnp.max(v, axis=1, keepdims=True)
            first = jnp.min(jnp.where(v == m, iota, S), axis=1, keepdims=True)
            v = jnp.where(iota == first, -1.0, v)
            tot = tot + m
        nt_ref[...] = jnp.maximum(v, 0.0)
        sc_ref[...] = jnp.broadcast_to(tot / TOPK, (B, 128))

    return pl.pallas_call(
        _topk_body,
        out_shape=(
            jax.ShapeDtypeStruct((B, S), jnp.float32),
            jax.ShapeDtypeStruct((B, 128), jnp.float32),
        ),
    )


def _final_body(b_ref, c_ref, w_ref, s_ref, o_ref):
    cnt = c_ref[0, 0]
    steer = s_ref[0, 0, 0] * w_ref[0, 0]
    mask = (cnt > 0.0).astype(jnp.float32)
    o_ref[0] = b_ref[0] + mask[:, None] * steer[None, :]


@functools.lru_cache(maxsize=None)
def _make_final(B, S, H, BS):
    nblk = S // BS
    return pl.pallas_call(
        _final_body,
        grid=(B, nblk),
        in_specs=[
            pl.BlockSpec((1, BS, H), lambda b, j: (b, j, 0)),
            pl.BlockSpec((1, 1, BS), lambda b, j: (b * nblk + j, 0, 0)),
            pl.BlockSpec((1, 1, H), lambda b, j: (b, 0, 0)),
            pl.BlockSpec((1, 1, 128), lambda b, j: (b, 0, 0)),
        ],
        out_specs=pl.BlockSpec((1, BS, H), lambda b, j: (b, j, 0)),
        out_shape=jax.ShapeDtypeStruct((B, S, H), jnp.float32),
    )


def kernel(base, intervention_positions, batch_weights):
    B, S, H = base.shape
    BS1 = 512
    BS3 = 1024
    nblk3 = S // BS3

    d = _make_matvec(B, S, H, BS1)(base, batch_weights)

    det, cnt = _make_sc_gather(B, S)(d, intervention_positions)

    nt, scale = _make_topk(B, S)(det)

    out = _make_final(B, S, H, BS3)(
        base,
        cnt.reshape(B * nblk3, 1, BS3),
        batch_weights,
        scale.reshape(B, 1, 128),
    )
    return out, det, nt


# speedup vs baseline: 1.2038x; 1.0047x over previous
import functools

import jax
import jax.numpy as jnp
from jax import lax
from jax.experimental import pallas as pl
from jax.experimental.pallas import tpu as pltpu
from jax.experimental.pallas import tpu_sc as plsc

TOPK = 32


def _matvec_body(b_ref, w_ref, d_ref):
    bb = b_ref[...].astype(jnp.bfloat16).astype(jnp.float32)
    ww = w_ref[...].astype(jnp.bfloat16).astype(jnp.float32)
    d_ref[...] = jnp.sum(bb * ww, axis=2)


@functools.lru_cache(maxsize=None)
def _make_matvec(B, S, H, BS):
    nblk = S // BS
    return pl.pallas_call(
        _matvec_body,
        grid=(nblk,),
        in_specs=[
            pl.BlockSpec((B, BS, H), lambda j: (0, j, 0)),
            pl.BlockSpec((B, 1, H), lambda j: (0, 0, 0)),
        ],
        out_specs=pl.BlockSpec((B, BS), lambda j: (0, j)),
        out_shape=jax.ShapeDtypeStruct((B, S), jnp.float32),
    )


@functools.lru_cache(maxsize=None)
def _make_sc_gather(B, S):
    info = plsc.get_sparse_core_info()
    NC, NS = info.num_cores, info.num_subcores
    NW = NC * NS
    per_b = NW // B
    b_per_c = B // NC
    CH = S // per_b
    R = CH // 128
    mesh = plsc.VectorSubcoreMesh(core_axis_name="c", subcore_axis_name="s")

    @functools.partial(
        pl.kernel,
        out_type=(
            jax.ShapeDtypeStruct((B, S), jnp.float32),
            jax.ShapeDtypeStruct((B, S), jnp.float32),
        ),
        mesh=mesh,
        compiler_params=pltpu.CompilerParams(needs_layout_passes=False),
        scratch_types=[
            pltpu.VMEM((S,), jnp.float32),
            pltpu.VMEM((CH,), jnp.int32),
            pltpu.VMEM((CH,), jnp.float32),
            pltpu.VMEM((R, 128), jnp.int32),
            pltpu.VMEM((128,), jnp.float32),
            pltpu.VMEM((CH,), jnp.float32),
            pltpu.VMEM_SHARED((b_per_c * S,), jnp.float32),
        ],
    )
    def sc_fn(d_hbm, pos_hbm, det_hbm, cnt_hbm,
              d_v, pos_v, det_v, idx_v, ones_v, tmp_v, cnt_sh):
        c = lax.axis_index("c")
        s = lax.axis_index("s")
        lb = s // per_b
        b = c * b_per_c + lb
        base_s = (s - lb * per_b) * CH

        pltpu.sync_copy(d_hbm.at[b], d_v)
        pltpu.sync_copy(pos_hbm.at[b, pl.ds(base_s, CH)], pos_v)

        one16 = jnp.ones((16,), jnp.float32)
        zero16 = jnp.zeros((16,), jnp.float32)
        boff = lb * S
        for i in range(8):
            ones_v[pl.ds(i * 16, 16)] = one16
        for i in range(CH // 16):
            sl = pl.ds(i * 16, 16)
            idx = pos_v[sl]
            val = plsc.load_gather(d_v, [idx])
            det_v[sl] = jnp.maximum(val, 0.0)
            idx_v[i // 8, pl.ds((i % 8) * 16, 16)] = idx + boff
            tmp_v[sl] = zero16

        pltpu.sync_copy(tmp_v, cnt_sh.at[pl.ds(s * CH, CH)])
        plsc.subcore_barrier()
        for r in range(R):
            pltpu.sync_copy(ones_v, cnt_sh.at[idx_v.at[r]], add=True)
        plsc.subcore_barrier()
        pltpu.sync_copy(cnt_sh.at[pl.ds(s * CH, CH)], tmp_v)
        pltpu.sync_copy(tmp_v, cnt_hbm.at[b, pl.ds(base_s, CH)])
        pltpu.sync_copy(det_v, det_hbm.at[b, pl.ds(base_s, CH)])

    return sc_fn


@functools.lru_cache(maxsize=None)
def _make_topk(B, S):
    def _topk_body(det_ref, nt_ref, sc_ref):
        v = det_ref[...]
        iota = lax.broadcasted_iota(jnp.int32, (B, S), 1)
        tot = jnp.zeros((B, 1), jnp.float32)
        for _ in range(TOPK):
            m = jnp.max(v, axis=1, keepdims=True)
            first = jnp.min(jnp.where(v == m, iota, S), axis=1, keepdims=True)
            v = jnp.where(iota == first, -1.0, v)
            tot = tot + m
        nt_ref[...] = jnp.maximum(v, 0.0)
        sc_ref[...] = jnp.broadcast_to(tot / TOPK, (B, 128))

    return pl.pallas_call(
        _topk_body,
        out_shape=(
            jax.ShapeDtypeStruct((B, S), jnp.float32),
            jax.ShapeDtypeStruct((B, 128), jnp.float32),
        ),
    )


def _final_body(b_ref, c_ref, w_ref, s_ref, o_ref):
    steer = s_ref[:, 0:1] * w_ref[:, 0, :]
    mask = (c_ref[...] > 0.0).astype(jnp.float32)
    o_ref[...] = b_ref[...] + mask[:, :, None] * steer[:, None, :]


@functools.lru_cache(maxsize=None)
def _make_final(B, S, H, BS):
    nblk = S // BS
    return pl.pallas_call(
        _final_body,
        grid=(nblk,),
        in_specs=[
            pl.BlockSpec((B, BS, H), lambda j: (0, j, 0)),
            pl.BlockSpec((B, BS), lambda j: (0, j)),
            pl.BlockSpec((B, 1, H), lambda j: (0, 0, 0)),
            pl.BlockSpec((B, 128), lambda j: (0, 0)),
        ],
        out_specs=pl.BlockSpec((B, BS, H), lambda j: (0, j, 0)),
        out_shape=jax.ShapeDtypeStruct((B, S, H), jnp.float32),
    )


def kernel(base, intervention_positions, batch_weights):
    B, S, H = base.shape
    BS1 = 512
    BS3 = 256

    d = _make_matvec(B, S, H, BS1)(base, batch_weights)

    det, cnt = _make_sc_gather(B, S)(d, intervention_positions)

    nt, scale = _make_topk(B, S)(det)

    out = _make_final(B, S, H, BS3)(base, cnt, batch_weights, scale)
    return out, det, nt
